# Initial kernel scaffold; baseline (speedup 1.0000x reference)
import jax
import jax.numpy as jnp
from jax.experimental import pallas as pl


def kernel(x, dim):
    raise NotImplementedError("write your pallas kernel here")


# speedup vs baseline: 3.1826x; 3.1826x over previous
import functools

import jax
import jax.numpy as jnp
from jax import lax
from jax.experimental import pallas as pl
from jax.experimental.pallas import tpu as pltpu
from jax.experimental.pallas import tpu_sc as plsc

R = 128
N = 32768
K = 512
L = 16
NBIN = 256
CH = N // L


def _key16(v):
    b = lax.bitcast_convert_type(v, jnp.int32)
    m = lax.shift_right_arithmetic(b, 31)
    k = lax.bitwise_xor(b, lax.bitwise_and(m, jnp.int32(0x7FFFFFFF)))
    return jnp.where(b == jnp.int32(-2147483648), jnp.int32(0), k)


def _build():
    info = plsc.get_sparse_core_info()
    nc, ns = info.num_cores, info.num_subcores
    nw = nc * ns
    rows_per_w = R // nw
    mesh = plsc.VectorSubcoreMesh(core_axis_name="c", subcore_axis_name="s")

    @functools.partial(
        pl.kernel,
        mesh=mesh,
        out_type=jax.ShapeDtypeStruct((R, K), jnp.float32),
        compiler_params=pltpu.CompilerParams(needs_layout_passes=False),
        scratch_types=[
            pltpu.VMEM((N,), jnp.float32),
            pltpu.VMEM((NBIN * L,), jnp.int32),
            pltpu.SMEM((NBIN,), jnp.int32),
            pltpu.VMEM((K + L,), jnp.float32),
        ],
    )
    def kmax(x_hbm, o_hbm, row_v, hist_v, tot_s, out_v):
        wid = lax.axis_index("s") * nc + lax.axis_index("c")
        iota = lax.iota(jnp.int32, L)
        ones = jnp.ones((L,), jnp.int32)
        zeros = jnp.zeros((L,), jnp.int32)

        def clear_hist(i, c):
            hist_v[pl.ds(i * L, L)] = zeros
            return c

        lax.fori_loop(0, NBIN, clear_hist, 0)

        def do_row(j, c):
            row = wid * rows_per_w + j
            pltpu.sync_copy(x_hbm.at[row], row_v)

            def scan0(i, c):
                k = _key16(row_v[pl.ds(i * L, L)])
                b = lax.bitwise_xor(
                    lax.bitwise_and(lax.shift_right_arithmetic(k, 24),
                                    jnp.int32(255)),
                    jnp.int32(128))
                idx = lax.shift_left(b, 4) + iota
                plsc.addupdate_scatter(hist_v, [idx], ones)
                return c

            lax.fori_loop(0, CH, scan0, 0)

            def totals(i, c):
                tot_s[i] = jnp.sum(hist_v[pl.ds(i * L, L)])
                hist_v[pl.ds(i * L, L)] = zeros
                return c

            def find_bin(i, carry):
                rem, bsel, found = carry
                b = NBIN - 1 - i
                cnt = tot_s[b]
                take = (found == 0) & (cnt >= rem)
                return (jnp.where((found == 0) & (cnt < rem), rem - cnt, rem),
                        jnp.where(take, b, bsel),
                        jnp.where(take, jnp.int32(1), found))

            lax.fori_loop(0, NBIN, totals, 0)
            rem, b0, _ = lax.fori_loop(
                0, NBIN, find_bin,
                (jnp.int32(K), jnp.int32(0), jnp.int32(0)))
            pv = lax.bitwise_xor(b0, jnp.int32(128))

            def refine(args):
                rem, pv, msh, bsh, mmask = args

                def scan(i, c):
                    k = _key16(row_v[pl.ds(i * L, L)])
                    mval = lax.bitwise_and(
                        lax.shift_right_arithmetic(k, msh), jnp.int32(mmask))
                    mask = mval == pv
                    b = lax.bitwise_and(
                        lax.shift_right_arithmetic(k, bsh), jnp.int32(255))
                    idx = lax.shift_left(b, 4) + iota
                    plsc.addupdate_scatter(hist_v, [idx], ones, mask=mask)
                    return c

                lax.fori_loop(0, CH, scan, 0)
                lax.fori_loop(0, NBIN, totals, 0)
                rem2, b2, _ = lax.fori_loop(
                    0, NBIN, find_bin, (rem, jnp.int32(0), jnp.int32(0)))
                return rem2, lax.bitwise_or(lax.shift_left(pv, 8), b2)

            rem, pv = refine((rem, pv, 24, 16, 0xFF))
            rem, pv = refine((rem, pv, 16, 8, 0xFFFF))
            rem, pv = refine((rem, pv, 8, 0, 0xFFFFFF))

            t = pv
            m = rem

            def compact(i, carry):
                ptr, tiec = carry
                v = row_v[pl.ds(i * L, L)]
                k = _key16(v)
                gt = k > t
                eq = k == t
                eqi = jnp.where(eq, jnp.int32(1), jnp.int32(0))
                exc = plsc.cumsum(eqi) - eqi
                sel = gt | (eq & ((exc + tiec) < m))
                plsc.store_compressed(out_v.at[pl.ds(ptr, L)], v, mask=sel)
                seli = jnp.where(sel, jnp.int32(1), jnp.int32(0))
                return (ptr + jnp.sum(seli), tiec + jnp.sum(eqi))

            lax.fori_loop(0, CH, compact, (jnp.int32(0), jnp.int32(0)))
            pltpu.sync_copy(out_v.at[pl.ds(0, K)], o_hbm.at[row])
            return c

        lax.fori_loop(0, rows_per_w, do_row, 0)

    return kmax


_kmax = _build()


def kernel(x, dim):
    del dim
    return _kmax(x)


# speedup vs baseline: 7.7452x; 2.4336x over previous
import functools

import jax
import jax.numpy as jnp
from jax import lax
from jax.experimental import pallas as pl
from jax.experimental.pallas import tpu as pltpu
from jax.experimental.pallas import tpu_sc as plsc

R = 128
N = 32768
K = 512
L = 16
NBIN = 256
CH = N // L


def _key16(v):
    b = lax.bitcast_convert_type(v, jnp.int32)
    m = lax.shift_right_arithmetic(b, 31)
    k = lax.bitwise_xor(b, lax.bitwise_and(m, jnp.int32(0x7FFFFFFF)))
    return jnp.where(b == jnp.int32(-2147483648), jnp.int32(0), k)


def _build():
    info = plsc.get_sparse_core_info()
    nc, ns = info.num_cores, info.num_subcores
    nw = nc * ns
    rows_per_w = R // nw
    mesh = plsc.VectorSubcoreMesh(core_axis_name="c", subcore_axis_name="s")

    @functools.partial(
        pl.kernel,
        mesh=mesh,
        out_type=jax.ShapeDtypeStruct((R, K), jnp.float32),
        compiler_params=pltpu.CompilerParams(needs_layout_passes=False),
        scratch_types=[
            pltpu.VMEM((N,), jnp.float32),
            pltpu.VMEM((N + L,), jnp.float32),
            pltpu.VMEM((NBIN * L,), jnp.int32),
            pltpu.VMEM((L * L,), jnp.int32),
            pltpu.SMEM((NBIN,), jnp.int32),
            pltpu.VMEM((K + L,), jnp.float32),
        ],
    )
    def kmax(x_hbm, o_hbm, row_v, cand_v, hist_v, hist4_v, tot_s, out_v):
        wid = lax.axis_index("s") * nc + lax.axis_index("c")
        iota = lax.iota(jnp.int32, L)
        ones = jnp.ones((L,), jnp.int32)
        zeros = jnp.zeros((L,), jnp.int32)

        def clear_hist(i, c):
            hist_v[pl.ds(i * L, L)] = zeros
            return c

        lax.fori_loop(0, NBIN, clear_hist, 0)

        def clear_hist4(i, c):
            hist4_v[pl.ds(i * L, L)] = zeros
            return c

        lax.fori_loop(0, L, clear_hist4, 0)

        def do_row(j, c):
            row = wid * rows_per_w + j
            pltpu.sync_copy(x_hbm.at[row], row_v)

            def scan0(i, c):
                k = _key16(row_v[pl.ds(i * L, L)])
                b = lax.bitwise_xor(
                    lax.bitwise_and(lax.shift_right_arithmetic(k, 24),
                                    jnp.int32(255)),
                    jnp.int32(128))
                idx = lax.shift_left(b, 4) + iota
                plsc.addupdate_scatter(hist_v, [idx], ones)
                return c

            lax.fori_loop(0, CH, scan0, 0)

            def totals(i, c):
                tot_s[i] = jnp.sum(hist_v[pl.ds(i * L, L)])
                hist_v[pl.ds(i * L, L)] = zeros
                return c

            def find_bin(i, carry):
                rem, bsel, found = carry
                b = NBIN - 1 - i
                cnt = tot_s[b]
                take = (found == 0) & (cnt >= rem)
                return (jnp.where((found == 0) & (cnt < rem), rem - cnt, rem),
                        jnp.where(take, b, bsel),
                        jnp.where(take, jnp.int32(1), found))

            lax.fori_loop(0, NBIN, totals, 0)
            rem, b0, _ = lax.fori_loop(
                0, NBIN, find_bin,
                (jnp.int32(K), jnp.int32(0), jnp.int32(0)))
            pv = lax.bitwise_xor(b0, jnp.int32(128))
            t_lo = lax.shift_left(pv, 24)

            def compact_cand(i, ptr):
                v = row_v[pl.ds(i * L, L)]
                sel = _key16(v) >= t_lo
                plsc.store_compressed(cand_v.at[pl.ds(ptr, L)], v, mask=sel)
                return ptr + jnp.sum(jnp.where(sel, jnp.int32(1),
                                               jnp.int32(0)))

            ncand = lax.fori_loop(0, CH, compact_cand, jnp.int32(0))
            ncc = lax.div(ncand + (L - 1), jnp.int32(L))

            def refine(rem, pv, rnd):
                msh = 24 - 4 * (rnd - 1)
                mmask = (1 << (8 + 4 * (rnd - 1))) - 1
                bsh = 24 - 4 * rnd

                def scan(i, c):
                    k = _key16(cand_v[pl.ds(i * L, L)])
                    mval = lax.bitwise_and(
                        lax.shift_right_arithmetic(k, msh), jnp.int32(mmask))
                    inb = (lax.shift_left(i, 4) + iota) < ncand
                    mask = (mval == pv) & inb
                    b = lax.bitwise_and(
                        lax.shift_right_arithmetic(k, bsh), jnp.int32(15))
                    idx = lax.shift_left(b, 4) + iota
                    plsc.addupdate_scatter(hist4_v, [idx], ones, mask=mask)
                    return c

                lax.fori_loop(0, ncc, scan, 0)

                def totals4(i, c):
                    tot_s[i] = jnp.sum(hist4_v[pl.ds(i * L, L)])
                    hist4_v[pl.ds(i * L, L)] = zeros
                    return c

                def find4(i, carry):
                    remc, bsel, found = carry
                    b = L - 1 - i
                    cnt = tot_s[b]
                    take = (found == 0) & (cnt >= remc)
                    return (jnp.where((found == 0) & (cnt < remc),
                                      remc - cnt, remc),
                            jnp.where(take, b, bsel),
                            jnp.where(take, jnp.int32(1), found))

                lax.fori_loop(0, L, totals4, 0)
                rem2, b2, _ = lax.fori_loop(
                    0, L, find4, (rem, jnp.int32(0), jnp.int32(0)))
                return rem2, lax.bitwise_or(lax.shift_left(pv, 4), b2)

            for rnd in range(1, 7):
                rem, pv = refine(rem, pv, rnd)

            t = pv
            m = rem

            def emit(i, carry):
                ptr, tiec = carry
                v = cand_v[pl.ds(i * L, L)]
                k = _key16(v)
                inb = (lax.shift_left(i, 4) + iota) < ncand
                gt = (k > t) & inb
                eq = (k == t) & inb
                eqi = jnp.where(eq, jnp.int32(1), jnp.int32(0))
                exc = plsc.cumsum(eqi) - eqi
                sel = gt | (eq & ((exc + tiec) < m))
                plsc.store_compressed(out_v.at[pl.ds(ptr, L)], v, mask=sel)
                seli = jnp.where(sel, jnp.int32(1), jnp.int32(0))
                return (ptr + jnp.sum(seli), tiec + jnp.sum(eqi))

            lax.fori_loop(0, ncc, emit, (jnp.int32(0), jnp.int32(0)))
            pltpu.sync_copy(out_v.at[pl.ds(0, K)], o_hbm.at[row])
            return c

        lax.fori_loop(0, rows_per_w, do_row, 0)

    return kmax


_kmax = _build()


def kernel(x, dim):
    del dim
    return _kmax(x)
